# baseline (speedup 1.0000x reference)
import functools

import jax
import jax.numpy as jnp
from jax import lax
from jax.experimental import pallas as pl
from jax.experimental.pallas import tpu as pltpu
from jax.experimental.pallas import tpu_sc as plsc

N, D, E, FF, TOPK = 2048, 1024, 8, 4096, 2
CAP = 320
STRIDE = 352
S = E * STRIDE
LANES = 128
TB = 256
FF_BLK = 512
NJ = FF // FF_BLK
ZROW = N
PAD_SLOT = CAP
SB = 256

NC, NS = 2, 16
NW = NC * NS
DISP_PER_TILE = S // NW
TOK_PER_TILE = N // NW
CHUNK_TOK = 32


def _router_body(x_ref, gwt_ref, idx_ref, w_ref, ps_ref):
    li = lax.broadcasted_iota(jnp.int32, (TB, LANES), 1)
    logits = jnp.dot(x_ref[...], gwt_ref[...],
                     preferred_element_type=jnp.float32,
                     precision=lax.Precision.DEFAULT)
    logits = jnp.where(li < E, logits, -1e30)
    m = jnp.max(logits, axis=1, keepdims=True)
    p = jnp.exp(logits - m)
    p = jnp.where(li < E, p, 0.0)
    probs = p / jnp.sum(p, axis=1, keepdims=True)
    m1 = jnp.max(probs, axis=1, keepdims=True)
    a1 = jnp.min(jnp.where(probs == m1, li, LANES), axis=1, keepdims=True)
    probs2 = jnp.where(li == a1, -1.0, probs)
    m2 = jnp.max(probs2, axis=1, keepdims=True)
    a2 = jnp.min(jnp.where(probs2 == m2, li, LANES), axis=1, keepdims=True)
    tot = m1 + m2
    idx_ref[...] = jnp.where(li == 0, a1, a2)
    w_ref[...] = jnp.where(li == 0, m1 / tot, m2 / tot)
    ps_ref[...] = jnp.sum(probs, axis=0, keepdims=True)[None]


_router_call = pl.pallas_call(
    _router_body,
    grid=(N // TB,),
    in_specs=[pl.BlockSpec((TB, D), lambda i: (i, 0)),
              pl.BlockSpec((D, LANES), lambda i: (0, 0))],
    out_specs=[pl.BlockSpec((TB, LANES), lambda i: (i, 0)),
               pl.BlockSpec((TB, LANES), lambda i: (i, 0)),
               pl.BlockSpec((1, 1, LANES), lambda i: (i, 0, 0))],
    out_shape=[jax.ShapeDtypeStruct((N, LANES), jnp.int32),
               jax.ShapeDtypeStruct((N, LANES), jnp.float32),
               jax.ShapeDtypeStruct((N // TB, 1, LANES), jnp.float32)],
)


def _route_body(a_ref, w_ref, ps_ref,
                src_ref, sw_ref, c0_ref, c1_ref, aux_ref):
    li = lax.broadcasted_iota(jnp.int32, (N, LANES), 1)
    A = a_ref[...]
    e0 = A[:, 0:1]
    e1 = A[:, 1:2]
    oh0 = (li == e0).astype(jnp.float32)
    oh1 = (li == e1).astype(jnp.float32)
    msum = oh0 + oh1

    r = lax.broadcasted_iota(jnp.int32, (128, 128), 0)
    c = lax.broadcasted_iota(jnp.int32, (128, 128), 1)
    ls = (r > c).astype(jnp.float32)
    carry = jnp.zeros((1, LANES), jnp.float32)
    parts = []
    for i in range(N // 128):
        ch = msum[128 * i:128 * (i + 1)]
        parts.append(jnp.dot(ls, ch, preferred_element_type=jnp.float32)
                     + carry)
        carry = carry + jnp.sum(ch, axis=0, keepdims=True)
    prefix = jnp.concatenate(parts, axis=0)
    counts = carry

    rank0 = jnp.sum(prefix * oh0, axis=1, keepdims=True) + 1.0
    rank1 = jnp.sum((prefix + oh0) * oh1, axis=1, keepdims=True) + 1.0
    r0i = rank0.astype(jnp.int32)
    r1i = rank1.astype(jnp.int32)
    slot0 = e0 * STRIDE + r0i - 1
    slot1 = e1 * STRIDE + r1i - 1
    v0 = r0i <= CAP
    v1 = r1i <= CAP
    c0_ref[...] = jnp.broadcast_to(jnp.where(v0, slot0, PAD_SLOT), (N, LANES))
    c1_ref[...] = jnp.broadcast_to(jnp.where(v1, slot1, PAD_SLOT), (N, LANES))

    wt = w_ref[...]

    def _split(w):
        hi = w.astype(jnp.bfloat16).astype(jnp.float32)
        return hi, w - hi

    w0h, w0l = _split(wt[:, 0:1])
    w1h, w1l = _split(wt[:, 1:2])
    tok = lax.broadcasted_iota(jnp.int32, (N, 1), 0)
    tokh = (tok // 256).astype(jnp.float32)
    tokl = (tok % 256).astype(jnp.float32)
    sm0 = jnp.where(v0, slot0, -1)
    sm1 = jnp.where(v1, slot1, -1)

    def _cols(wh, wl):
        return jnp.where(
            li == 0, tokh,
            jnp.where(li == 1, tokl,
                      jnp.where(li == 2, 1.0,
                                jnp.where(li == 3, wh,
                                          jnp.where(li == 4, wl, 0.0)))))

    cols0 = _cols(w0h, w0l)
    cols1 = _cols(w1h, w1l)
    dn = (((0,), (0,)), ((), ()))
    for b in range(S // SB):
        sl = lax.broadcasted_iota(jnp.int32, (N, SB), 1) + b * SB
        m0 = (sl == sm0).astype(jnp.float32)
        m1_ = (sl == sm1).astype(jnp.float32)
        res = (lax.dot_general(m0, cols0, dn,
                               preferred_element_type=jnp.float32)
               + lax.dot_general(m1_, cols1, dn,
                                 preferred_element_type=jnp.float32))
        fil = res[:, 2:3]
        src_f = res[:, 0:1] * 256.0 + res[:, 1:2]
        src_v = jnp.where(fil > 0.5, src_f.astype(jnp.int32), ZROW)
        w_v = jnp.where(fil > 0.5, res[:, 3:4] + res[:, 4:5], 0.0)
        src_ref[b * SB:(b + 1) * SB, :] = jnp.broadcast_to(src_v, (SB, LANES))
        sw_ref[b * SB:(b + 1) * SB, :] = jnp.broadcast_to(w_v, (SB, LANES))

    pbar = jnp.sum(ps_ref[...], axis=0) / N
    f = counts / (N * TOPK + 1e-9)
    aux = E * jnp.sum(f * pbar, axis=1, keepdims=True) * 0.01
    aux_ref[...] = jnp.broadcast_to(aux, (8, LANES))


_route_call = pl.pallas_call(
    _route_body,
    out_shape=[jax.ShapeDtypeStruct((S, LANES), jnp.int32),
               jax.ShapeDtypeStruct((S, LANES), jnp.float32),
               jax.ShapeDtypeStruct((N, LANES), jnp.int32),
               jax.ShapeDtypeStruct((N, LANES), jnp.int32),
               jax.ShapeDtypeStruct((8, LANES), jnp.float32)],
)


def _ffn_body(x_ref, wg_ref, wu_ref, wd_ref, sw_ref, o_ref, acc_ref):
    j = pl.program_id(1)
    xb = x_ref[...]
    g = jnp.dot(xb, wg_ref[0], preferred_element_type=jnp.float32)
    u = jnp.dot(xb, wu_ref[0], preferred_element_type=jnp.float32)
    h = g * jax.nn.sigmoid(g) * u
    part = jnp.dot(h, wd_ref[0], preferred_element_type=jnp.float32)

    @pl.when(j == 0)
    def _():
        acc_ref[...] = part

    @pl.when(j > 0)
    def _():
        acc_ref[...] += part

    @pl.when(j == NJ - 1)
    def _():
        o_ref[...] = acc_ref[...] * sw_ref[...]


_ffn_call = pl.pallas_call(
    _ffn_body,
    grid=(E, NJ),
    in_specs=[pl.BlockSpec((STRIDE, D), lambda e, j: (e, 0)),
              pl.BlockSpec((1, D, FF_BLK), lambda e, j: (e, 0, j)),
              pl.BlockSpec((1, D, FF_BLK), lambda e, j: (e, 0, j)),
              pl.BlockSpec((1, FF_BLK, D), lambda e, j: (e, j, 0)),
              pl.BlockSpec((STRIDE, 1), lambda e, j: (e, 0))],
    out_specs=pl.BlockSpec((STRIDE, D), lambda e, j: (e, 0)),
    out_shape=jax.ShapeDtypeStruct((S, D), jnp.float32),
    scratch_shapes=[pltpu.VMEM((STRIDE, D), jnp.float32)],
)


@functools.cache
def _sc_kernels():
    mesh = plsc.VectorSubcoreMesh(core_axis_name="c", subcore_axis_name="s",
                                  num_cores=NC, num_subcores=NS)

    @functools.partial(
        pl.kernel,
        out_type=jax.ShapeDtypeStruct((S, D), jnp.float32),
        mesh=mesh,
        scratch_types=[pltpu.VMEM((DISP_PER_TILE,), jnp.int32),
                       pltpu.VMEM((DISP_PER_TILE, D), jnp.float32),
                       pltpu.SemaphoreType.DMA,
                       pltpu.SemaphoreType.DMA],
    )
    def dispatch(x_hbm, idx_hbm, out_hbm, idx_v, rows_v, sem_g, sem_w):
        wid = lax.axis_index("s") * NC + lax.axis_index("c")
        base = wid * DISP_PER_TILE
        pltpu.sync_copy(idx_hbm.at[pl.ds(base, DISP_PER_TILE)], idx_v)
        pltpu.async_copy(x_hbm.at[idx_v], rows_v, sem_g).wait()
        pltpu.async_copy(
            rows_v, out_hbm.at[pl.ds(base, DISP_PER_TILE)], sem_w).wait()

    nv = D // 16
    npass = TOK_PER_TILE // CHUNK_TOK

    @functools.partial(
        pl.kernel,
        out_type=jax.ShapeDtypeStruct((N, D), jnp.float32),
        mesh=mesh,
        scratch_types=[pltpu.VMEM((2 * TOK_PER_TILE,), jnp.int32),
                       pltpu.VMEM((2 * CHUNK_TOK, D), jnp.float32),
                       pltpu.VMEM((CHUNK_TOK, D), jnp.float32),
                       pltpu.SemaphoreType.DMA,
                       pltpu.SemaphoreType.DMA],
    )
    def combine(eo_hbm, idx_hbm, out_hbm, idx_v, g_v, o_v, sem_g, sem_w):
        wid = lax.axis_index("s") * NC + lax.axis_index("c")
        ibase = wid * 2 * TOK_PER_TILE
        obase = wid * TOK_PER_TILE
        pltpu.sync_copy(idx_hbm.at[pl.ds(ibase, 2 * TOK_PER_TILE)], idx_v)
        g = pltpu.async_copy(
            eo_hbm.at[idx_v.at[pl.ds(0, 2 * CHUNK_TOK)]], g_v, sem_g)
        w_prev = None
        for p in range(npass):
            g.wait()
            if w_prev is not None:
                w_prev.wait()

            @plsc.parallel_loop(0, CHUNK_TOK * nv, unroll=8)
            def _(i):
                row = i // nv
                off = (i % nv) * 16
                o_v[row, pl.ds(off, 16)] = (
                    g_v[2 * row, pl.ds(off, 16)]
                    + g_v[2 * row + 1, pl.ds(off, 16)])

            if p + 1 < npass:
                g = pltpu.async_copy(
                    eo_hbm.at[idx_v.at[pl.ds((p + 1) * 2 * CHUNK_TOK,
                                             2 * CHUNK_TOK)]],
                    g_v, sem_g)
            w_prev = pltpu.async_copy(
                o_v, out_hbm.at[pl.ds(obase + p * CHUNK_TOK, CHUNK_TOK)],
                sem_w)
        w_prev.wait()

    return dispatch, combine


def kernel(x, gate_w, w_gate, w_up, w_down):
    x_flat = x.reshape(N, D)
    gwt = jnp.zeros((D, LANES), jnp.float32).at[:, :E].set(gate_w.T)
    idx_pad, w_pad, ps = _router_call(x_flat, gwt)
    src, sw, c0, c1, aux = _route_call(idx_pad, w_pad, ps)
    x_ext = jnp.zeros((N + 8, D), jnp.float32).at[:N].set(x_flat)
    dispatch, combine = _sc_kernels()
    disp = dispatch(x_ext, src[:, 0])
    eo = _ffn_call(disp, w_gate, w_up, w_down, sw[:, 0:1])
    c01 = jnp.stack([c0[:, 0], c1[:, 0]], axis=1).reshape(-1)
    out = combine(eo, c01)
    return out.reshape(1, N, D), aux[0, 0]


# speedup vs baseline: 1.0360x; 1.0360x over previous
import functools

import jax
import jax.numpy as jnp
from jax import lax
from jax.experimental import pallas as pl
from jax.experimental.pallas import tpu as pltpu
from jax.experimental.pallas import tpu_sc as plsc

N, D, E, FF, TOPK = 2048, 1024, 8, 4096, 2
CAP = 320
STRIDE = 352
S = E * STRIDE
LANES = 128
TB = 256
FF_BLK = 512
NJ = FF // FF_BLK
ZROW = 0
PAD_SLOT = CAP
SB = 256

NC, NS = 2, 16
NW = NC * NS
DISP_PER_TILE = S // NW
TOK_PER_TILE = N // NW
CHUNK_TOK = 32


def _route_body(x_ref, gwt_ref,
                src_ref, sw_ref, c0_ref, c1_ref, aux_ref,
                a_s, w_s, ps_s):
    i = pl.program_id(0)
    lib = lax.broadcasted_iota(jnp.int32, (TB, LANES), 1)
    logits = jnp.dot(x_ref[...], gwt_ref[...],
                     preferred_element_type=jnp.float32,
                     precision=lax.Precision.DEFAULT)
    logits = jnp.where(lib < E, logits, -1e30)
    m = jnp.max(logits, axis=1, keepdims=True)
    p = jnp.exp(logits - m)
    p = jnp.where(lib < E, p, 0.0)
    probs = p / jnp.sum(p, axis=1, keepdims=True)
    m1 = jnp.max(probs, axis=1, keepdims=True)
    a1 = jnp.min(jnp.where(probs == m1, lib, LANES), axis=1, keepdims=True)
    probs2 = jnp.where(lib == a1, -1.0, probs)
    m2 = jnp.max(probs2, axis=1, keepdims=True)
    a2 = jnp.min(jnp.where(probs2 == m2, lib, LANES), axis=1, keepdims=True)
    tot = m1 + m2
    a_s[pl.ds(i * TB, TB), :] = jnp.where(lib == 0, a1, a2)
    w_s[pl.ds(i * TB, TB), :] = jnp.where(lib == 0, m1 / tot, m2 / tot)
    psum = jnp.sum(probs, axis=0, keepdims=True)

    @pl.when(i == 0)
    def _():
        ps_s[...] = psum

    @pl.when(i > 0)
    def _():
        ps_s[...] += psum

    @pl.when(i == N // TB - 1)
    def _():
        _rank_and_invert(a_s, w_s, ps_s,
                         src_ref, sw_ref, c0_ref, c1_ref, aux_ref)


def _rank_and_invert(a_ref, w_ref, ps_ref,
                     src_ref, sw_ref, c0_ref, c1_ref, aux_ref):
    li = lax.broadcasted_iota(jnp.int32, (N, LANES), 1)
    A = a_ref[...]
    e0 = A[:, 0:1]
    e1 = A[:, 1:2]
    oh0 = (li == e0).astype(jnp.float32)
    oh1 = (li == e1).astype(jnp.float32)
    msum = oh0 + oh1

    r = lax.broadcasted_iota(jnp.int32, (128, 128), 0)
    c = lax.broadcasted_iota(jnp.int32, (128, 128), 1)
    ls = (r > c).astype(jnp.float32)
    carry = jnp.zeros((1, LANES), jnp.float32)
    parts = []
    for i in range(N // 128):
        ch = msum[128 * i:128 * (i + 1)]
        parts.append(jnp.dot(ls, ch, preferred_element_type=jnp.float32)
                     + carry)
        carry = carry + jnp.sum(ch, axis=0, keepdims=True)
    prefix = jnp.concatenate(parts, axis=0)
    counts = carry

    rank0 = jnp.sum(prefix * oh0, axis=1, keepdims=True) + 1.0
    rank1 = jnp.sum((prefix + oh0) * oh1, axis=1, keepdims=True) + 1.0
    r0i = rank0.astype(jnp.int32)
    r1i = rank1.astype(jnp.int32)
    slot0 = e0 * STRIDE + r0i - 1
    slot1 = e1 * STRIDE + r1i - 1
    v0 = r0i <= CAP
    v1 = r1i <= CAP
    c0_ref[...] = jnp.broadcast_to(jnp.where(v0, slot0, PAD_SLOT), (N, LANES))
    c1_ref[...] = jnp.broadcast_to(jnp.where(v1, slot1, PAD_SLOT), (N, LANES))

    wt = w_ref[...]

    def _split(w):
        hi = w.astype(jnp.bfloat16).astype(jnp.float32)
        return hi, w - hi

    w0h, w0l = _split(wt[:, 0:1])
    w1h, w1l = _split(wt[:, 1:2])
    tok = lax.broadcasted_iota(jnp.int32, (N, 1), 0)
    tokh = (tok // 256).astype(jnp.float32)
    tokl = (tok % 256).astype(jnp.float32)
    sm0 = jnp.where(v0, slot0, -1)
    sm1 = jnp.where(v1, slot1, -1)

    def _cols(wh, wl):
        return jnp.where(
            li == 0, tokh,
            jnp.where(li == 1, tokl,
                      jnp.where(li == 2, 1.0,
                                jnp.where(li == 3, wh,
                                          jnp.where(li == 4, wl, 0.0)))))

    cols0 = _cols(w0h, w0l)
    cols1 = _cols(w1h, w1l)
    dn = (((0,), (0,)), ((), ()))
    for b in range(S // SB):
        sl = lax.broadcasted_iota(jnp.int32, (N, SB), 1) + b * SB
        m0 = (sl == sm0).astype(jnp.float32)
        m1_ = (sl == sm1).astype(jnp.float32)
        res = (lax.dot_general(m0, cols0, dn,
                               preferred_element_type=jnp.float32)
               + lax.dot_general(m1_, cols1, dn,
                                 preferred_element_type=jnp.float32))
        fil = res[:, 2:3]
        src_f = res[:, 0:1] * 256.0 + res[:, 1:2]
        src_v = jnp.where(fil > 0.5, src_f.astype(jnp.int32), ZROW)
        w_v = jnp.where(fil > 0.5, res[:, 3:4] + res[:, 4:5], 0.0)
        src_ref[b * SB:(b + 1) * SB, :] = jnp.broadcast_to(src_v, (SB, LANES))
        sw_ref[b * SB:(b + 1) * SB, :] = jnp.broadcast_to(w_v, (SB, LANES))

    pbar = ps_ref[...] / N
    f = counts / (N * TOPK + 1e-9)
    aux = E * jnp.sum(f * pbar, axis=1, keepdims=True) * 0.01
    aux_ref[...] = jnp.broadcast_to(aux, (8, LANES))


_route_call = pl.pallas_call(
    _route_body,
    grid=(N // TB,),
    in_specs=[pl.BlockSpec((TB, D), lambda i: (i, 0)),
              pl.BlockSpec((D, LANES), lambda i: (0, 0))],
    out_specs=[pl.BlockSpec((S, LANES), lambda i: (0, 0)),
               pl.BlockSpec((S, LANES), lambda i: (0, 0)),
               pl.BlockSpec((N, LANES), lambda i: (0, 0)),
               pl.BlockSpec((N, LANES), lambda i: (0, 0)),
               pl.BlockSpec((8, LANES), lambda i: (0, 0))],
    out_shape=[jax.ShapeDtypeStruct((S, LANES), jnp.int32),
               jax.ShapeDtypeStruct((S, LANES), jnp.float32),
               jax.ShapeDtypeStruct((N, LANES), jnp.int32),
               jax.ShapeDtypeStruct((N, LANES), jnp.int32),
               jax.ShapeDtypeStruct((8, LANES), jnp.float32)],
    scratch_shapes=[pltpu.VMEM((N, LANES), jnp.int32),
                    pltpu.VMEM((N, LANES), jnp.float32),
                    pltpu.VMEM((1, LANES), jnp.float32)],
)


def _ffn_body(x_ref, wg_ref, wu_ref, wd_ref, sw_ref, o_ref, acc_ref):
    j = pl.program_id(1)
    xb = x_ref[...]
    g = jnp.dot(xb, wg_ref[0], preferred_element_type=jnp.float32)
    u = jnp.dot(xb, wu_ref[0], preferred_element_type=jnp.float32)
    h = g * jax.nn.sigmoid(g) * u
    part = jnp.dot(h, wd_ref[0], preferred_element_type=jnp.float32)

    @pl.when(j == 0)
    def _():
        acc_ref[...] = part

    @pl.when(j > 0)
    def _():
        acc_ref[...] += part

    @pl.when(j == NJ - 1)
    def _():
        o_ref[...] = acc_ref[...] * sw_ref[...]


_ffn_call = pl.pallas_call(
    _ffn_body,
    grid=(E, NJ),
    in_specs=[pl.BlockSpec((STRIDE, D), lambda e, j: (e, 0)),
              pl.BlockSpec((1, D, FF_BLK), lambda e, j: (e, 0, j)),
              pl.BlockSpec((1, D, FF_BLK), lambda e, j: (e, 0, j)),
              pl.BlockSpec((1, FF_BLK, D), lambda e, j: (e, j, 0)),
              pl.BlockSpec((STRIDE, 1), lambda e, j: (e, 0))],
    out_specs=pl.BlockSpec((STRIDE, D), lambda e, j: (e, 0)),
    out_shape=jax.ShapeDtypeStruct((S, D), jnp.float32),
    scratch_shapes=[pltpu.VMEM((STRIDE, D), jnp.float32)],
)


@functools.cache
def _sc_kernels():
    mesh = plsc.VectorSubcoreMesh(core_axis_name="c", subcore_axis_name="s",
                                  num_cores=NC, num_subcores=NS)

    @functools.partial(
        pl.kernel,
        out_type=jax.ShapeDtypeStruct((S, D), jnp.float32),
        mesh=mesh,
        scratch_types=[pltpu.VMEM((DISP_PER_TILE,), jnp.int32),
                       pltpu.VMEM((DISP_PER_TILE, D), jnp.float32),
                       pltpu.SemaphoreType.DMA,
                       pltpu.SemaphoreType.DMA],
    )
    def dispatch(x_hbm, idx_hbm, out_hbm, idx_v, rows_v, sem_g, sem_w):
        wid = lax.axis_index("s") * NC + lax.axis_index("c")
        base = wid * DISP_PER_TILE
        pltpu.sync_copy(idx_hbm.at[pl.ds(base, DISP_PER_TILE)], idx_v)
        pltpu.async_copy(x_hbm.at[idx_v], rows_v, sem_g).wait()
        pltpu.async_copy(
            rows_v, out_hbm.at[pl.ds(base, DISP_PER_TILE)], sem_w).wait()

    nv = D // 16
    npass = TOK_PER_TILE // CHUNK_TOK

    @functools.partial(
        pl.kernel,
        out_type=jax.ShapeDtypeStruct((N, D), jnp.float32),
        mesh=mesh,
        scratch_types=[pltpu.VMEM((2 * TOK_PER_TILE,), jnp.int32),
                       pltpu.VMEM((2 * CHUNK_TOK, D), jnp.float32),
                       pltpu.VMEM((CHUNK_TOK, D), jnp.float32),
                       pltpu.SemaphoreType.DMA,
                       pltpu.SemaphoreType.DMA],
    )
    def combine(eo_hbm, idx_hbm, out_hbm, idx_v, g_v, o_v, sem_g, sem_w):
        wid = lax.axis_index("s") * NC + lax.axis_index("c")
        ibase = wid * 2 * TOK_PER_TILE
        obase = wid * TOK_PER_TILE
        pltpu.sync_copy(idx_hbm.at[pl.ds(ibase, 2 * TOK_PER_TILE)], idx_v)
        g = pltpu.async_copy(
            eo_hbm.at[idx_v.at[pl.ds(0, 2 * CHUNK_TOK)]], g_v, sem_g)
        w_prev = None
        for p in range(npass):
            g.wait()
            if w_prev is not None:
                w_prev.wait()

            @plsc.parallel_loop(0, CHUNK_TOK * nv, unroll=8)
            def _(i):
                row = i // nv
                off = (i % nv) * 16
                o_v[row, pl.ds(off, 16)] = (
                    g_v[2 * row, pl.ds(off, 16)]
                    + g_v[2 * row + 1, pl.ds(off, 16)])

            if p + 1 < npass:
                g = pltpu.async_copy(
                    eo_hbm.at[idx_v.at[pl.ds((p + 1) * 2 * CHUNK_TOK,
                                             2 * CHUNK_TOK)]],
                    g_v, sem_g)
            w_prev = pltpu.async_copy(
                o_v, out_hbm.at[pl.ds(obase + p * CHUNK_TOK, CHUNK_TOK)],
                sem_w)
        w_prev.wait()

    return dispatch, combine


def kernel(x, gate_w, w_gate, w_up, w_down):
    x_flat = x.reshape(N, D)
    gwt = jnp.zeros((D, LANES), jnp.float32).at[:, :E].set(gate_w.T)
    src, sw, c0, c1, aux = _route_call(x_flat, gwt)
    dispatch, combine = _sc_kernels()
    disp = dispatch(x_flat, src[:, 0])
    eo = _ffn_call(disp, w_gate, w_up, w_down, sw[:, 0:1])
    c01 = jnp.stack([c0[:, 0], c1[:, 0]], axis=1).reshape(-1)
    out = combine(eo, c01)
    return out.reshape(1, N, D), aux[0, 0]
